# Initial kernel scaffold; baseline (speedup 1.0000x reference)
import jax
import jax.numpy as jnp
from jax.experimental import pallas as pl


def kernel(x, edge_index, edge_features, params):
    raise NotImplementedError("write your pallas kernel here")


# speedup vs baseline: 1.8862x; 1.8862x over previous
import jax
import jax.numpy as jnp
from jax import lax
from jax.experimental import pallas as pl
from jax.experimental.pallas import tpu as pltpu
from jax.experimental.pallas import tpu_sc as plsc

_N = 10000
_E = 160000
_L = 128
_DE = 16
_STEPS = 5

_NC, _NS = 2, 16
_NW = _NC * _NS
_CHUNK = 128
_CPT = 40
_EP = _NW * _CPT * _CHUNK
_ROWS = 10112
_RPT = _ROWS // _NS

_BE = 1024
_BN = 1000


def _dot(a, b):
    return jnp.dot(a, b, preferred_element_type=jnp.float32)


def _ln(z, g, b):
    mu = jnp.mean(z, axis=-1, keepdims=True)
    zc = z - mu
    var = jnp.mean(zc * zc, axis=-1, keepdims=True)
    return zc * lax.rsqrt(var + 1e-5) * g + b


def _full(shape):
    return pl.BlockSpec(shape, lambda i: (0,) * len(shape))


def _rows(bs, ncols):
    return pl.BlockSpec((bs, ncols), lambda i: (i, 0))


def _edge_enc_body(ef, w0, b0, w1, b1, w2, b2, g, b, out):
    z = jnp.maximum(_dot(ef[...], w0[...]) + b0[...], 0.0)
    z = jnp.maximum(_dot(z, w1[...]) + b1[...], 0.0)
    z = _dot(z, w2[...]) + b2[...]
    out[...] = _ln(z, g[...], b[...])


def _edge_enc(efp, w0, b0, w1, b1, w2, b2, g, b):
    return pl.pallas_call(
        _edge_enc_body,
        grid=(_EP // _BE,),
        in_specs=[_rows(_BE, _DE), _full((_DE, _L)), _full((1, _L)),
                  _full((_L, _L)), _full((1, _L)), _full((_L, _L)),
                  _full((1, _L)), _full((1, _L)), _full((1, _L))],
        out_specs=_rows(_BE, _L),
        out_shape=jax.ShapeDtypeStruct((_EP, _L), jnp.float32),
    )(efp, w0, b0, w1, b1, w2, b2, g, b)


def _node_enc_body(x, w0, b0, w1, b1, w2, b2, g, b, wa, wb,
                   h_out, ha_out, hb_out):
    z = jnp.maximum(_dot(x[...], w0[...]) + b0[...], 0.0)
    z = jnp.maximum(_dot(z, w1[...]) + b1[...], 0.0)
    z = _dot(z, w2[...]) + b2[...]
    h = _ln(z, g[...], b[...])
    h_out[...] = h
    ha_out[...] = _dot(h, wa[...])
    hb_out[...] = _dot(h, wb[...])


def _node_enc(x, w0, b0, w1, b1, w2, b2, g, b, wa, wb):
    sds = jax.ShapeDtypeStruct((_N, _L), jnp.float32)
    return pl.pallas_call(
        _node_enc_body,
        grid=(_N // _BN,),
        in_specs=[_rows(_BN, _L)] + [_full((_L, _L)), _full((1, _L))] * 3
                 + [_full((1, _L)), _full((1, _L)),
                    _full((_L, _L)), _full((_L, _L))],
        out_specs=[_rows(_BN, _L)] * 3,
        out_shape=[sds, sds, sds],
    )(x, w0, b0, w1, b1, w2, b2, g, b, wa, wb)


def _edge_step_body(ga, gb, e, w1c, b1, w2, b2, w3, b3, g, b,
                    enew_out, eout_out):
    t = jnp.maximum(ga[...] + gb[...] + _dot(e[...], w1c[...]) + b1[...], 0.0)
    t = jnp.maximum(_dot(t, w2[...]) + b2[...], 0.0)
    t = _dot(t, w3[...]) + b3[...]
    en = _ln(t, g[...], b[...])
    enew_out[...] = en
    eout_out[...] = e[...] + en


def _edge_step(ga, gb, e, w1c, b1, w2, b2, w3, b3, g, b):
    sds = jax.ShapeDtypeStruct((_EP, _L), jnp.float32)
    return pl.pallas_call(
        _edge_step_body,
        grid=(_EP // _BE,),
        in_specs=[_rows(_BE, _L)] * 3
                 + [_full((_L, _L)), _full((1, _L))] * 3
                 + [_full((1, _L)), _full((1, _L))],
        out_specs=[_rows(_BE, _L)] * 2,
        out_shape=[sds, sds],
    )(ga, gb, e, w1c, b1, w2, b2, w3, b3, g, b)


def _node_step_body(a0, a1, h, va, c1, vb, v2, c2, v3, c3, g, b, wa, wb,
                    h_out, ha_out, hb_out):
    a = a0[...] + a1[...]
    t = jnp.maximum(_dot(a, va[...]) + _dot(h[...], vb[...]) + c1[...], 0.0)
    t = jnp.maximum(_dot(t, v2[...]) + c2[...], 0.0)
    t = _dot(t, v3[...]) + c3[...]
    hn = _ln(t, g[...], b[...])
    ho = h[...] + hn
    h_out[...] = ho
    ha_out[...] = _dot(ho, wa[...])
    hb_out[...] = _dot(ho, wb[...])


def _node_step(a0, a1, h, va, c1, vb, v2, c2, v3, c3, g, b, wa, wb):
    sds = jax.ShapeDtypeStruct((_N, _L), jnp.float32)
    return pl.pallas_call(
        _node_step_body,
        grid=(_N // _BN,),
        in_specs=[_rows(_BN, _L)] * 3
                 + [_full((_L, _L)), _full((1, _L)), _full((_L, _L)),
                    _full((_L, _L)), _full((1, _L)),
                    _full((_L, _L)), _full((1, _L))]
                 + [_full((1, _L)), _full((1, _L)),
                    _full((_L, _L)), _full((_L, _L))],
        out_specs=[_rows(_BN, _L)] * 3,
        out_shape=[sds, sds, sds],
    )(a0, a1, h, va, c1, vb, v2, c2, v3, c3, g, b, wa, wb)


def _dec_body(h, w0, b0, w1, b1, w2, b2, out):
    z = jnp.maximum(_dot(h[...], w0[...]) + b0[...], 0.0)
    z = jnp.maximum(_dot(z, w1[...]) + b1[...], 0.0)
    out[...] = _dot(z, w2[...]) + b2[...]


def _dec(h, w0, b0, w1, b1, w2p, b2p):
    return pl.pallas_call(
        _dec_body,
        grid=(_N // _BN,),
        in_specs=[_rows(_BN, _L)] + [_full((_L, _L)), _full((1, _L))] * 3,
        out_specs=_rows(_BN, _L),
        out_shape=jax.ShapeDtypeStruct((_N, _L), jnp.float32),
    )(h, w0, b0, w1, b1, w2p, b2p)


def _gather_body(ha, hb, dstp, srcp, ga, gb, idx_d, idx_s, buf_a, buf_b,
                 sem_a, sem_b):
    wid = lax.axis_index("s") * _NC + lax.axis_index("c")
    base0 = wid * (_CPT * _CHUNK)

    def chunk(i, carry):
        base = base0 + i * _CHUNK
        pltpu.sync_copy(dstp.at[pl.ds(base, _CHUNK)], idx_d)
        pltpu.sync_copy(srcp.at[pl.ds(base, _CHUNK)], idx_s)
        ca = pltpu.async_copy(ha.at[idx_d], buf_a, sem_a)
        cb = pltpu.async_copy(hb.at[idx_s], buf_b, sem_b)
        ca.wait()
        cb.wait()
        pltpu.sync_copy(buf_a, ga.at[pl.ds(base, _CHUNK)])
        pltpu.sync_copy(buf_b, gb.at[pl.ds(base, _CHUNK)])
        return carry

    lax.fori_loop(0, _CPT, chunk, 0)


def _scatter_body(enew, dstp, zeros, agg, idx, buf, acc):
    cid = lax.axis_index("c")
    sid = lax.axis_index("s")
    wid = sid * _NC + cid
    base0 = wid * (_CPT * _CHUNK)

    pltpu.sync_copy(zeros.at[pl.ds(sid * _RPT, _RPT)],
                    acc.at[pl.ds(sid * _RPT, _RPT)])
    plsc.subcore_barrier()

    def chunk(i, carry):
        base = base0 + i * _CHUNK
        pltpu.sync_copy(dstp.at[pl.ds(base, _CHUNK)], idx)
        pltpu.sync_copy(enew.at[pl.ds(base, _CHUNK)], buf)
        pltpu.sync_copy(buf, acc.at[idx], add=True)
        return carry

    lax.fori_loop(0, _CPT, chunk, 0)
    plsc.subcore_barrier()

    pltpu.sync_copy(acc.at[pl.ds(sid * _RPT, _RPT)],
                    agg.at[cid, pl.ds(sid * _RPT, _RPT)])


_SC_CALLS = {}


def _sc_calls():
    if not _SC_CALLS:
        mesh = plsc.VectorSubcoreMesh(core_axis_name="c", subcore_axis_name="s",
                                      num_cores=_NC, num_subcores=_NS)
        _SC_CALLS['gather'] = pl.kernel(
            _gather_body,
            out_type=[jax.ShapeDtypeStruct((_EP, _L), jnp.float32)] * 2,
            mesh=mesh,
            scratch_types=[
                pltpu.VMEM((_CHUNK,), jnp.int32),
                pltpu.VMEM((_CHUNK,), jnp.int32),
                pltpu.VMEM((_CHUNK, _L), jnp.float32),
                pltpu.VMEM((_CHUNK, _L), jnp.float32),
                pltpu.SemaphoreType.DMA,
                pltpu.SemaphoreType.DMA,
            ],
        )
        _SC_CALLS['scatter'] = pl.kernel(
            _scatter_body,
            out_type=jax.ShapeDtypeStruct((_NC, _ROWS, _L), jnp.float32),
            mesh=mesh,
            scratch_types=[
                pltpu.VMEM((_CHUNK,), jnp.int32),
                pltpu.VMEM((_CHUNK, _L), jnp.float32),
                pltpu.VMEM_SHARED((_ROWS, _L), jnp.float32),
            ],
        )
    return _SC_CALLS


def _sc_gather(ha, hb, dst_g, src_g):
    return _sc_calls()['gather'](ha, hb, dst_g, src_g)


def _sc_scatter(enew, dst_s, zeros):
    return _sc_calls()['scatter'](enew, dst_s, zeros)


def kernel(x, edge_index, edge_features, params):
    src = edge_index[0].astype(jnp.int32)
    dst = edge_index[1].astype(jnp.int32)
    pad = _EP - _E
    dst_g = jnp.concatenate([dst, jnp.zeros((pad,), jnp.int32)])
    src_g = jnp.concatenate([src, jnp.zeros((pad,), jnp.int32)])
    dst_s = jnp.concatenate([dst, jnp.full((pad,), _N, jnp.int32)])
    efp = jnp.concatenate(
        [edge_features, jnp.zeros((pad, _DE), jnp.float32)], axis=0)
    zeros = jnp.zeros((_ROWS, _L), jnp.float32)

    def r1(v):
        return v.reshape(1, -1)

    (enc_n_mlp, enc_n_ln) = params['enc_node']
    (enc_e_mlp, enc_e_ln) = params['enc_edge']
    inets = params['inets']

    w1a = [p['edge_mlp'][0][0][:_L] for p in inets]
    w1b = [p['edge_mlp'][0][0][_L:2 * _L] for p in inets]
    w1c = [p['edge_mlp'][0][0][2 * _L:] for p in inets]
    va = [p['node_mlp'][0][0][:_L] for p in inets]
    vb = [p['node_mlp'][0][0][_L:] for p in inets]

    e = _edge_enc(efp,
                  enc_e_mlp[0][0], r1(enc_e_mlp[0][1]),
                  enc_e_mlp[1][0], r1(enc_e_mlp[1][1]),
                  enc_e_mlp[2][0], r1(enc_e_mlp[2][1]),
                  r1(enc_e_ln[0]), r1(enc_e_ln[1]))
    h, ha, hb = _node_enc(x,
                          enc_n_mlp[0][0], r1(enc_n_mlp[0][1]),
                          enc_n_mlp[1][0], r1(enc_n_mlp[1][1]),
                          enc_n_mlp[2][0], r1(enc_n_mlp[2][1]),
                          r1(enc_n_ln[0]), r1(enc_n_ln[1]),
                          w1a[0], w1b[0])

    for s in range(_STEPS):
        p = inets[s]
        ga, gb = _sc_gather(ha, hb, dst_g, src_g)
        enew, e = _edge_step(ga, gb, e,
                             w1c[s], r1(p['edge_mlp'][0][1]),
                             p['edge_mlp'][1][0], r1(p['edge_mlp'][1][1]),
                             p['edge_mlp'][2][0], r1(p['edge_mlp'][2][1]),
                             r1(p['edge_ln'][0]), r1(p['edge_ln'][1]))
        aggp = _sc_scatter(enew, dst_s, zeros)
        nxt = (s + 1) % _STEPS
        h, ha, hb = _node_step(aggp[0, :_N], aggp[1, :_N], h,
                               va[s], r1(p['node_mlp'][0][1]),
                               vb[s],
                               p['node_mlp'][1][0], r1(p['node_mlp'][1][1]),
                               p['node_mlp'][2][0], r1(p['node_mlp'][2][1]),
                               r1(p['node_ln'][0]), r1(p['node_ln'][1]),
                               w1a[nxt], w1b[nxt])

    dec = params['dec']
    w2p = jnp.zeros((_L, _L), jnp.float32).at[:, :3].set(dec[2][0])
    b2p = jnp.zeros((1, _L), jnp.float32).at[0, :3].set(dec[2][1])
    y = _dec(h, dec[0][0], r1(dec[0][1]), dec[1][0], r1(dec[1][1]), w2p, b2p)
    return y[:, :3]


# speedup vs baseline: 2.2809x; 1.2092x over previous
import jax
import jax.numpy as jnp
from jax import lax
from jax.experimental import pallas as pl
from jax.experimental.pallas import tpu as pltpu
from jax.experimental.pallas import tpu_sc as plsc

_N = 10000
_E = 160000
_L = 128
_DE = 16
_STEPS = 5

_NC, _NS = 2, 16
_NW = _NC * _NS
_CHUNK = 128
_CPT = 40
_EP = _NW * _CPT * _CHUNK
_ROWS = 10112
_RPT = _ROWS // _NS

_BE = 1024
_BN = 1000


def _dot(a, b):
    return jnp.dot(a, b, preferred_element_type=jnp.float32)


def _ln(z, g, b):
    mu = jnp.mean(z, axis=-1, keepdims=True)
    zc = z - mu
    var = jnp.mean(zc * zc, axis=-1, keepdims=True)
    return zc * lax.rsqrt(var + 1e-5) * g + b


def _full(shape):
    return pl.BlockSpec(shape, lambda i: (0,) * len(shape))


def _rows(bs, ncols):
    return pl.BlockSpec((bs, ncols), lambda i: (i, 0))


def _edge_enc_body(ef, w0, b0, w1, b1, w2, b2, g, b, out):
    z = jnp.maximum(_dot(ef[...], w0[...]) + b0[...], 0.0)
    z = jnp.maximum(_dot(z, w1[...]) + b1[...], 0.0)
    z = _dot(z, w2[...]) + b2[...]
    out[...] = _ln(z, g[...], b[...])


def _edge_enc(efp, w0, b0, w1, b1, w2, b2, g, b):
    return pl.pallas_call(
        _edge_enc_body,
        grid=(_EP // _BE,),
        in_specs=[_rows(_BE, _DE), _full((_DE, _L)), _full((1, _L)),
                  _full((_L, _L)), _full((1, _L)), _full((_L, _L)),
                  _full((1, _L)), _full((1, _L)), _full((1, _L))],
        out_specs=_rows(_BE, _L),
        out_shape=jax.ShapeDtypeStruct((_EP, _L), jnp.float32),
    )(efp, w0, b0, w1, b1, w2, b2, g, b)


def _node_enc_body(x, w0, b0, w1, b1, w2, b2, g, b, wa, wb,
                   h_out, ha_out, hb_out):
    z = jnp.maximum(_dot(x[...], w0[...]) + b0[...], 0.0)
    z = jnp.maximum(_dot(z, w1[...]) + b1[...], 0.0)
    z = _dot(z, w2[...]) + b2[...]
    h = _ln(z, g[...], b[...])
    h_out[...] = h
    ha_out[...] = _dot(h, wa[...])
    hb_out[...] = _dot(h, wb[...])


def _node_enc(x, w0, b0, w1, b1, w2, b2, g, b, wa, wb):
    sds = jax.ShapeDtypeStruct((_N, _L), jnp.float32)
    return pl.pallas_call(
        _node_enc_body,
        grid=(_N // _BN,),
        in_specs=[_rows(_BN, _L)] + [_full((_L, _L)), _full((1, _L))] * 3
                 + [_full((1, _L)), _full((1, _L)),
                    _full((_L, _L)), _full((_L, _L))],
        out_specs=[_rows(_BN, _L)] * 3,
        out_shape=[sds, sds, sds],
    )(x, w0, b0, w1, b1, w2, b2, g, b, wa, wb)


def _edge_step_body(ga, gb, e, w1c, b1, w2, b2, w3, b3, g, b,
                    enew_out, eout_out):
    t = jnp.maximum(ga[...] + gb[...] + _dot(e[...], w1c[...]) + b1[...], 0.0)
    t = jnp.maximum(_dot(t, w2[...]) + b2[...], 0.0)
    t = _dot(t, w3[...]) + b3[...]
    en = _ln(t, g[...], b[...])
    enew_out[...] = en
    eout_out[...] = e[...] + en


def _edge_step(ga, gb, e, w1c, b1, w2, b2, w3, b3, g, b):
    sds = jax.ShapeDtypeStruct((_EP, _L), jnp.float32)
    return pl.pallas_call(
        _edge_step_body,
        grid=(_EP // _BE,),
        in_specs=[_rows(_BE, _L)] * 3
                 + [_full((_L, _L)), _full((1, _L))] * 3
                 + [_full((1, _L)), _full((1, _L))],
        out_specs=[_rows(_BE, _L)] * 2,
        out_shape=[sds, sds],
    )(ga, gb, e, w1c, b1, w2, b2, w3, b3, g, b)


def _node_step_body(a0, a1, h, va, c1, vb, v2, c2, v3, c3, g, b, wa, wb,
                    h_out, ha_out, hb_out):
    a = a0[...] + a1[...]
    t = jnp.maximum(_dot(a, va[...]) + _dot(h[...], vb[...]) + c1[...], 0.0)
    t = jnp.maximum(_dot(t, v2[...]) + c2[...], 0.0)
    t = _dot(t, v3[...]) + c3[...]
    hn = _ln(t, g[...], b[...])
    ho = h[...] + hn
    h_out[...] = ho
    ha_out[...] = _dot(ho, wa[...])
    hb_out[...] = _dot(ho, wb[...])


def _node_step(a0, a1, h, va, c1, vb, v2, c2, v3, c3, g, b, wa, wb):
    sds = jax.ShapeDtypeStruct((_N, _L), jnp.float32)
    return pl.pallas_call(
        _node_step_body,
        grid=(_N // _BN,),
        in_specs=[_rows(_BN, _L)] * 3
                 + [_full((_L, _L)), _full((1, _L)), _full((_L, _L)),
                    _full((_L, _L)), _full((1, _L)),
                    _full((_L, _L)), _full((1, _L))]
                 + [_full((1, _L)), _full((1, _L)),
                    _full((_L, _L)), _full((_L, _L))],
        out_specs=[_rows(_BN, _L)] * 3,
        out_shape=[sds, sds, sds],
    )(a0, a1, h, va, c1, vb, v2, c2, v3, c3, g, b, wa, wb)


def _dec_body(h, w0, b0, w1, b1, w2, b2, out):
    z = jnp.maximum(_dot(h[...], w0[...]) + b0[...], 0.0)
    z = jnp.maximum(_dot(z, w1[...]) + b1[...], 0.0)
    out[...] = _dot(z, w2[...]) + b2[...]


def _dec(h, w0, b0, w1, b1, w2p, b2p):
    return pl.pallas_call(
        _dec_body,
        grid=(_N // _BN,),
        in_specs=[_rows(_BN, _L)] + [_full((_L, _L)), _full((1, _L))] * 3,
        out_specs=_rows(_BN, _L),
        out_shape=jax.ShapeDtypeStruct((_N, _L), jnp.float32),
    )(h, w0, b0, w1, b1, w2p, b2p)


_NB = 2


def _gather_body(ha, hb, dst2, src2, ga, gb, idxd, idxs,
                 bufa0, bufa1, bufb0, bufb1, sga0, sga1, sgb0, sgb1):
    wid = lax.axis_index("s") * _NC + lax.axis_index("c")
    cbase = wid * _CPT
    bufa, bufb = (bufa0, bufa1), (bufb0, bufb1)
    sga, sgb = (sga0, sga1), (sgb0, sgb1)

    pltpu.sync_copy(dst2.at[pl.ds(cbase, _CPT)], idxd)
    pltpu.sync_copy(src2.at[pl.ds(cbase, _CPT)], idxs)

    for b in range(_NB):
        pltpu.async_copy(ha.at[idxd.at[b]], bufa[b], sga[b])
        pltpu.async_copy(hb.at[idxs.at[b]], bufb[b], sgb[b])

    def drain(b, ci):
        grow = (cbase + ci) * _CHUNK
        pltpu.make_async_copy(ha.at[idxd.at[0]], bufa[b], sga[b]).wait()
        pltpu.make_async_copy(hb.at[idxs.at[0]], bufb[b], sgb[b]).wait()
        pltpu.sync_copy(bufa[b], ga.at[pl.ds(grow, _CHUNK)])
        pltpu.sync_copy(bufb[b], gb.at[pl.ds(grow, _CHUNK)])

    def pair(j, carry):
        for b in range(_NB):
            ci = j * _NB + b
            drain(b, ci)
            pltpu.async_copy(ha.at[idxd.at[ci + _NB]], bufa[b], sga[b])
            pltpu.async_copy(hb.at[idxs.at[ci + _NB]], bufb[b], sgb[b])
        return carry

    lax.fori_loop(0, _CPT // _NB - 1, pair, 0)
    for b in range(_NB):
        drain(b, _CPT - _NB + b)


def _scatter_body(enew, dst2, zeros, agg, idxa, buf0, buf1, sl0, sl1, acc):
    cid = lax.axis_index("c")
    sid = lax.axis_index("s")
    wid = sid * _NC + cid
    cbase = wid * _CPT
    bufs, sl = (buf0, buf1), (sl0, sl1)

    pltpu.sync_copy(zeros.at[pl.ds(sid * _RPT, _RPT)],
                    acc.at[pl.ds(sid * _RPT, _RPT)])
    pltpu.sync_copy(dst2.at[pl.ds(cbase, _CPT)], idxa)
    plsc.subcore_barrier()

    for b in range(_NB):
        pltpu.async_copy(enew.at[pl.ds((cbase + b) * _CHUNK, _CHUNK)],
                         bufs[b], sl[b])

    def add(b, ci):
        pltpu.make_async_copy(enew.at[pl.ds(0, _CHUNK)], bufs[b],
                              sl[b]).wait()
        pltpu.sync_copy(bufs[b], acc.at[idxa.at[ci]], add=True)

    def pair(j, carry):
        for b in range(_NB):
            ci = j * _NB + b
            add(b, ci)
            pltpu.async_copy(
                enew.at[pl.ds((cbase + ci + _NB) * _CHUNK, _CHUNK)],
                bufs[b], sl[b])
        return carry

    lax.fori_loop(0, _CPT // _NB - 1, pair, 0)
    for b in range(_NB):
        add(b, _CPT - _NB + b)
    plsc.subcore_barrier()

    pltpu.sync_copy(acc.at[pl.ds(sid * _RPT, _RPT)],
                    agg.at[cid, pl.ds(sid * _RPT, _RPT)])


_SC_CALLS = {}


def _sc_calls():
    if not _SC_CALLS:
        mesh = plsc.VectorSubcoreMesh(core_axis_name="c", subcore_axis_name="s",
                                      num_cores=_NC, num_subcores=_NS)
        _SC_CALLS['gather'] = pl.kernel(
            _gather_body,
            out_type=[jax.ShapeDtypeStruct((_EP, _L), jnp.float32)] * 2,
            mesh=mesh,
            scratch_types=(
                [pltpu.VMEM((_CPT, _CHUNK), jnp.int32)] * 2
                + [pltpu.VMEM((_CHUNK, _L), jnp.float32)] * (2 * _NB)
                + [pltpu.SemaphoreType.DMA] * (2 * _NB)
            ),
        )
        _SC_CALLS['scatter'] = pl.kernel(
            _scatter_body,
            out_type=jax.ShapeDtypeStruct((_NC, _ROWS, _L), jnp.float32),
            mesh=mesh,
            scratch_types=(
                [pltpu.VMEM((_CPT, _CHUNK), jnp.int32)]
                + [pltpu.VMEM((_CHUNK, _L), jnp.float32)] * _NB
                + [pltpu.SemaphoreType.DMA] * _NB
                + [pltpu.VMEM_SHARED((_ROWS, _L), jnp.float32)]
            ),
        )
    return _SC_CALLS


def _sc_gather(ha, hb, dst_g, src_g):
    return _sc_calls()['gather'](ha, hb, dst_g, src_g)


def _sc_scatter(enew, dst_s, zeros):
    return _sc_calls()['scatter'](enew, dst_s, zeros)


def kernel(x, edge_index, edge_features, params):
    src = edge_index[0].astype(jnp.int32)
    dst = edge_index[1].astype(jnp.int32)
    pad = _EP - _E
    nrow = _NW * _CPT
    dst_g = jnp.concatenate([dst, jnp.zeros((pad,), jnp.int32)]
                            ).reshape(nrow, _CHUNK)
    src_g = jnp.concatenate([src, jnp.zeros((pad,), jnp.int32)]
                            ).reshape(nrow, _CHUNK)
    dst_s = jnp.concatenate([dst, jnp.full((pad,), _N, jnp.int32)]
                            ).reshape(nrow, _CHUNK)
    efp = jnp.concatenate(
        [edge_features, jnp.zeros((pad, _DE), jnp.float32)], axis=0)
    zeros = jnp.zeros((_ROWS, _L), jnp.float32)

    def r1(v):
        return v.reshape(1, -1)

    (enc_n_mlp, enc_n_ln) = params['enc_node']
    (enc_e_mlp, enc_e_ln) = params['enc_edge']
    inets = params['inets']

    w1a = [p['edge_mlp'][0][0][:_L] for p in inets]
    w1b = [p['edge_mlp'][0][0][_L:2 * _L] for p in inets]
    w1c = [p['edge_mlp'][0][0][2 * _L:] for p in inets]
    va = [p['node_mlp'][0][0][:_L] for p in inets]
    vb = [p['node_mlp'][0][0][_L:] for p in inets]

    e = _edge_enc(efp,
                  enc_e_mlp[0][0], r1(enc_e_mlp[0][1]),
                  enc_e_mlp[1][0], r1(enc_e_mlp[1][1]),
                  enc_e_mlp[2][0], r1(enc_e_mlp[2][1]),
                  r1(enc_e_ln[0]), r1(enc_e_ln[1]))
    h, ha, hb = _node_enc(x,
                          enc_n_mlp[0][0], r1(enc_n_mlp[0][1]),
                          enc_n_mlp[1][0], r1(enc_n_mlp[1][1]),
                          enc_n_mlp[2][0], r1(enc_n_mlp[2][1]),
                          r1(enc_n_ln[0]), r1(enc_n_ln[1]),
                          w1a[0], w1b[0])

    for s in range(_STEPS):
        p = inets[s]
        ga, gb = _sc_gather(ha, hb, dst_g, src_g)
        enew, e = _edge_step(ga, gb, e,
                             w1c[s], r1(p['edge_mlp'][0][1]),
                             p['edge_mlp'][1][0], r1(p['edge_mlp'][1][1]),
                             p['edge_mlp'][2][0], r1(p['edge_mlp'][2][1]),
                             r1(p['edge_ln'][0]), r1(p['edge_ln'][1]))
        aggp = _sc_scatter(enew, dst_s, zeros)
        nxt = (s + 1) % _STEPS
        h, ha, hb = _node_step(aggp[0, :_N], aggp[1, :_N], h,
                               va[s], r1(p['node_mlp'][0][1]),
                               vb[s],
                               p['node_mlp'][1][0], r1(p['node_mlp'][1][1]),
                               p['node_mlp'][2][0], r1(p['node_mlp'][2][1]),
                               r1(p['node_ln'][0]), r1(p['node_ln'][1]),
                               w1a[nxt], w1b[nxt])

    dec = params['dec']
    w2p = jnp.zeros((_L, _L), jnp.float32).at[:, :3].set(dec[2][0])
    b2p = jnp.zeros((1, _L), jnp.float32).at[0, :3].set(dec[2][1])
    y = _dec(h, dec[0][0], r1(dec[0][1]), dec[1][0], r1(dec[1][1]), w2p, b2p)
    return y[:, :3]


# speedup vs baseline: 3.5377x; 1.5510x over previous
import jax
import jax.numpy as jnp
from jax import lax
from jax.experimental import pallas as pl
from jax.experimental.pallas import tpu as pltpu
from jax.experimental.pallas import tpu_sc as plsc

_N = 10000
_E = 160000
_L = 128
_DE = 16
_STEPS = 5

_NC, _NS = 2, 16
_NW = _NC * _NS
_CHUNK = 40
_CPT = 125
_EP = _NW * _CPT * _CHUNK
_ROWS = 10112
_RPT = _ROWS // _NS

_BE = 1000
_BN = 1000


def _dot(a, b):
    return jnp.dot(a, b, preferred_element_type=jnp.float32)


def _ln(z, g, b):
    mu = jnp.mean(z, axis=-1, keepdims=True)
    zc = z - mu
    var = jnp.mean(zc * zc, axis=-1, keepdims=True)
    return zc * lax.rsqrt(var + 1e-5) * g + b


def _full(shape):
    return pl.BlockSpec(shape, lambda i: (0,) * len(shape))


def _rows(bs, ncols):
    return pl.BlockSpec((bs, ncols), lambda i: (i, 0))


def _edge_enc_body(ef, w0, b0, w1, b1, w2, b2, g, b, out):
    z = jnp.maximum(_dot(ef[...], w0[...]) + b0[...], 0.0)
    z = jnp.maximum(_dot(z, w1[...]) + b1[...], 0.0)
    z = _dot(z, w2[...]) + b2[...]
    out[...] = _ln(z, g[...], b[...])


def _edge_enc(efp, w0, b0, w1, b1, w2, b2, g, b):
    return pl.pallas_call(
        _edge_enc_body,
        grid=(_EP // _BE,),
        in_specs=[_rows(_BE, _DE), _full((_DE, _L)), _full((1, _L)),
                  _full((_L, _L)), _full((1, _L)), _full((_L, _L)),
                  _full((1, _L)), _full((1, _L)), _full((1, _L))],
        out_specs=_rows(_BE, _L),
        out_shape=jax.ShapeDtypeStruct((_EP, _L), jnp.float32),
    )(efp, w0, b0, w1, b1, w2, b2, g, b)


def _node_enc_body(x, w0, b0, w1, b1, w2, b2, g, b, wa, wb,
                   h_out, ha_out, hb_out):
    z = jnp.maximum(_dot(x[...], w0[...]) + b0[...], 0.0)
    z = jnp.maximum(_dot(z, w1[...]) + b1[...], 0.0)
    z = _dot(z, w2[...]) + b2[...]
    h = _ln(z, g[...], b[...])
    h_out[...] = h
    ha_out[...] = _dot(h, wa[...])
    hb_out[...] = _dot(h, wb[...])


def _node_enc(x, w0, b0, w1, b1, w2, b2, g, b, wa, wb):
    sds = jax.ShapeDtypeStruct((_N, _L), jnp.float32)
    return pl.pallas_call(
        _node_enc_body,
        grid=(_N // _BN,),
        in_specs=[_rows(_BN, _L)] + [_full((_L, _L)), _full((1, _L))] * 3
                 + [_full((1, _L)), _full((1, _L)),
                    _full((_L, _L)), _full((_L, _L))],
        out_specs=[_rows(_BN, _L)] * 3,
        out_shape=[sds, sds, sds],
    )(x, w0, b0, w1, b1, w2, b2, g, b, wa, wb)


def _edge_step_body(gsum, e, w1c, b1, w2, b2, w3, b3, g, b,
                    enew_out, eout_out):
    t = jnp.maximum(gsum[...] + _dot(e[...], w1c[...]) + b1[...], 0.0)
    t = jnp.maximum(_dot(t, w2[...]) + b2[...], 0.0)
    t = _dot(t, w3[...]) + b3[...]
    en = _ln(t, g[...], b[...])
    enew_out[...] = en
    eout_out[...] = e[...] + en


def _edge_step(gsum, e, w1c, b1, w2, b2, w3, b3, g, b):
    sds = jax.ShapeDtypeStruct((_EP, _L), jnp.float32)
    return pl.pallas_call(
        _edge_step_body,
        grid=(_EP // _BE,),
        in_specs=[_rows(_BE, _L)] * 2
                 + [_full((_L, _L)), _full((1, _L))] * 3
                 + [_full((1, _L)), _full((1, _L))],
        out_specs=[_rows(_BE, _L)] * 2,
        out_shape=[sds, sds],
    )(gsum, e, w1c, b1, w2, b2, w3, b3, g, b)


def _node_step_body(a0, a1, h, va, c1, vb, v2, c2, v3, c3, g, b, wa, wb,
                    h_out, ha_out, hb_out):
    a = a0[...] + a1[...]
    t = jnp.maximum(_dot(a, va[...]) + _dot(h[...], vb[...]) + c1[...], 0.0)
    t = jnp.maximum(_dot(t, v2[...]) + c2[...], 0.0)
    t = _dot(t, v3[...]) + c3[...]
    hn = _ln(t, g[...], b[...])
    ho = h[...] + hn
    h_out[...] = ho
    ha_out[...] = _dot(ho, wa[...])
    hb_out[...] = _dot(ho, wb[...])


def _node_step(a0, a1, h, va, c1, vb, v2, c2, v3, c3, g, b, wa, wb):
    sds = jax.ShapeDtypeStruct((_N, _L), jnp.float32)
    return pl.pallas_call(
        _node_step_body,
        grid=(_N // _BN,),
        in_specs=[_rows(_BN, _L)] * 3
                 + [_full((_L, _L)), _full((1, _L)), _full((_L, _L)),
                    _full((_L, _L)), _full((1, _L)),
                    _full((_L, _L)), _full((1, _L))]
                 + [_full((1, _L)), _full((1, _L)),
                    _full((_L, _L)), _full((_L, _L))],
        out_specs=[_rows(_BN, _L)] * 3,
        out_shape=[sds, sds, sds],
    )(a0, a1, h, va, c1, vb, v2, c2, v3, c3, g, b, wa, wb)


def _dec_body(h, w0, b0, w1, b1, w2, b2, out):
    z = jnp.maximum(_dot(h[...], w0[...]) + b0[...], 0.0)
    z = jnp.maximum(_dot(z, w1[...]) + b1[...], 0.0)
    out[...] = _dot(z, w2[...]) + b2[...]


def _dec(h, w0, b0, w1, b1, w2p, b2p):
    return pl.pallas_call(
        _dec_body,
        grid=(_N // _BN,),
        in_specs=[_rows(_BN, _L)] + [_full((_L, _L)), _full((1, _L))] * 3,
        out_specs=_rows(_BN, _L),
        out_shape=jax.ShapeDtypeStruct((_N, _L), jnp.float32),
    )(h, w0, b0, w1, b1, w2p, b2p)


_NB = 5


def _gather_body(ha, hb, dst3, src3, g, idxd, idxs,
                 bufa0, bufa1, bufa2, bufa3, bufa4,
                 bufb0, bufb1, bufb2, bufb3, bufb4,
                 sga0, sga1, sga2, sga3, sga4,
                 sgb0, sgb1, sgb2, sgb3, sgb4):
    wid = lax.axis_index("s") * _NC + lax.axis_index("c")
    cbase = wid * _CPT
    bufa = (bufa0, bufa1, bufa2, bufa3, bufa4)
    bufb = (bufb0, bufb1, bufb2, bufb3, bufb4)
    sga = (sga0, sga1, sga2, sga3, sga4)
    sgb = (sgb0, sgb1, sgb2, sgb3, sgb4)

    pltpu.sync_copy(dst3.at[wid], idxd)
    pltpu.sync_copy(src3.at[wid], idxs)

    for b in range(_NB):
        pltpu.async_copy(ha.at[idxd.at[b]], bufa[b], sga[b])
        pltpu.async_copy(hb.at[idxs.at[b]], bufb[b], sgb[b])

    def drain(b, ci):
        grow = (cbase + ci) * _CHUNK
        pltpu.make_async_copy(ha.at[idxd.at[0]], bufa[b], sga[b]).wait()
        pltpu.make_async_copy(hb.at[idxs.at[0]], bufb[b], sgb[b]).wait()

        def addrow(r, carry):
            for c in range(_L // 16):
                sl = pl.ds(c * 16, 16)
                bufa[b][r, sl] = bufa[b][r, sl] + bufb[b][r, sl]
            return carry

        lax.fori_loop(0, _CHUNK, addrow, 0)
        pltpu.sync_copy(bufa[b], g.at[pl.ds(grow, _CHUNK)])

    def ring(j, carry):
        for b in range(_NB):
            ci = j * _NB + b
            drain(b, ci)
            pltpu.async_copy(ha.at[idxd.at[ci + _NB]], bufa[b], sga[b])
            pltpu.async_copy(hb.at[idxs.at[ci + _NB]], bufb[b], sgb[b])
        return carry

    lax.fori_loop(0, _CPT // _NB - 1, ring, 0)
    for b in range(_NB):
        drain(b, _CPT - _NB + b)


def _scatter_body(enew, dst3, zeros, agg, idxa,
                  buf0, buf1, buf2, buf3, buf4,
                  sl0, sl1, sl2, sl3, sl4, acc):
    cid = lax.axis_index("c")
    sid = lax.axis_index("s")
    wid = sid * _NC + cid
    cbase = wid * _CPT
    bufs = (buf0, buf1, buf2, buf3, buf4)
    sl = (sl0, sl1, sl2, sl3, sl4)

    pltpu.sync_copy(zeros.at[pl.ds(sid * _RPT, _RPT)],
                    acc.at[pl.ds(sid * _RPT, _RPT)])
    pltpu.sync_copy(dst3.at[wid], idxa)
    plsc.subcore_barrier()

    for b in range(_NB):
        pltpu.async_copy(enew.at[pl.ds((cbase + b) * _CHUNK, _CHUNK)],
                         bufs[b], sl[b])

    def add(b, ci):
        pltpu.make_async_copy(enew.at[pl.ds(0, _CHUNK)], bufs[b],
                              sl[b]).wait()
        pltpu.sync_copy(bufs[b], acc.at[idxa.at[ci]], add=True)

    def pair(j, carry):
        for b in range(_NB):
            ci = j * _NB + b
            add(b, ci)
            pltpu.async_copy(
                enew.at[pl.ds((cbase + ci + _NB) * _CHUNK, _CHUNK)],
                bufs[b], sl[b])
        return carry

    lax.fori_loop(0, _CPT // _NB - 1, pair, 0)
    for b in range(_NB):
        add(b, _CPT - _NB + b)
    plsc.subcore_barrier()

    pltpu.sync_copy(acc.at[pl.ds(sid * _RPT, _RPT)],
                    agg.at[cid, pl.ds(sid * _RPT, _RPT)])


_SC_CALLS = {}


def _sc_calls():
    if not _SC_CALLS:
        mesh = plsc.VectorSubcoreMesh(core_axis_name="c", subcore_axis_name="s",
                                      num_cores=_NC, num_subcores=_NS)
        _SC_CALLS['gather'] = pl.kernel(
            _gather_body,
            out_type=jax.ShapeDtypeStruct((_EP, _L), jnp.float32),
            mesh=mesh,
            scratch_types=(
                [pltpu.VMEM((_CPT, _CHUNK), jnp.int32)] * 2
                + [pltpu.VMEM((_CHUNK, _L), jnp.float32)] * (2 * _NB)
                + [pltpu.SemaphoreType.DMA] * (2 * _NB)
            ),
        )
        _SC_CALLS['scatter'] = pl.kernel(
            _scatter_body,
            out_type=jax.ShapeDtypeStruct((_NC, _ROWS, _L), jnp.float32),
            mesh=mesh,
            scratch_types=(
                [pltpu.VMEM((_CPT, _CHUNK), jnp.int32)]
                + [pltpu.VMEM((_CHUNK, _L), jnp.float32)] * _NB
                + [pltpu.SemaphoreType.DMA] * _NB
                + [pltpu.VMEM_SHARED((_ROWS, _L), jnp.float32)]
            ),
        )
    return _SC_CALLS


def _sc_gather(ha, hb, dst_g, src_g):
    return _sc_calls()['gather'](ha, hb, dst_g, src_g)


def _sc_scatter(enew, dst_s, zeros):
    return _sc_calls()['scatter'](enew, dst_s, zeros)


def kernel(x, edge_index, edge_features, params):
    src = edge_index[0].astype(jnp.int32)
    dst = edge_index[1].astype(jnp.int32)
    dst_g = dst.reshape(_NW, _CPT, _CHUNK)
    src_g = src.reshape(_NW, _CPT, _CHUNK)
    efp = edge_features
    zeros = jnp.zeros((_ROWS, _L), jnp.float32)

    def r1(v):
        return v.reshape(1, -1)

    (enc_n_mlp, enc_n_ln) = params['enc_node']
    (enc_e_mlp, enc_e_ln) = params['enc_edge']
    inets = params['inets']

    w1a = [p['edge_mlp'][0][0][:_L] for p in inets]
    w1b = [p['edge_mlp'][0][0][_L:2 * _L] for p in inets]
    w1c = [p['edge_mlp'][0][0][2 * _L:] for p in inets]
    va = [p['node_mlp'][0][0][:_L] for p in inets]
    vb = [p['node_mlp'][0][0][_L:] for p in inets]

    e = _edge_enc(efp,
                  enc_e_mlp[0][0], r1(enc_e_mlp[0][1]),
                  enc_e_mlp[1][0], r1(enc_e_mlp[1][1]),
                  enc_e_mlp[2][0], r1(enc_e_mlp[2][1]),
                  r1(enc_e_ln[0]), r1(enc_e_ln[1]))
    h, ha, hb = _node_enc(x,
                          enc_n_mlp[0][0], r1(enc_n_mlp[0][1]),
                          enc_n_mlp[1][0], r1(enc_n_mlp[1][1]),
                          enc_n_mlp[2][0], r1(enc_n_mlp[2][1]),
                          r1(enc_n_ln[0]), r1(enc_n_ln[1]),
                          w1a[0], w1b[0])

    for s in range(_STEPS):
        p = inets[s]
        g = _sc_gather(ha, hb, dst_g, src_g)
        enew, e = _edge_step(g, e,
                             w1c[s], r1(p['edge_mlp'][0][1]),
                             p['edge_mlp'][1][0], r1(p['edge_mlp'][1][1]),
                             p['edge_mlp'][2][0], r1(p['edge_mlp'][2][1]),
                             r1(p['edge_ln'][0]), r1(p['edge_ln'][1]))
        aggp = _sc_scatter(enew, dst_g, zeros)
        nxt = (s + 1) % _STEPS
        h, ha, hb = _node_step(aggp[0, :_N], aggp[1, :_N], h,
                               va[s], r1(p['node_mlp'][0][1]),
                               vb[s],
                               p['node_mlp'][1][0], r1(p['node_mlp'][1][1]),
                               p['node_mlp'][2][0], r1(p['node_mlp'][2][1]),
                               r1(p['node_ln'][0]), r1(p['node_ln'][1]),
                               w1a[nxt], w1b[nxt])

    dec = params['dec']
    w2p = jnp.zeros((_L, _L), jnp.float32).at[:, :3].set(dec[2][0])
    b2p = jnp.zeros((1, _L), jnp.float32).at[0, :3].set(dec[2][1])
    y = _dec(h, dec[0][0], r1(dec[0][1]), dec[1][0], r1(dec[1][1]), w2p, b2p)
    return y[:, :3]
